# baseline (speedup 1.0000x reference)
import functools

import jax
import jax.numpy as jnp
from jax import lax
from jax.experimental import pallas as pl
from jax.experimental.pallas import tpu as pltpu
from jax.experimental.pallas import tpu_sc as plsc

MAX_LEN = 3000
EMBED_DIM = 32
BATCH = 16384
SEQ = 200

DT = EMBED_DIM // 8
NG = 8
PER_W = BATCH // NG
BT = PER_W // 128
NBUF = 4
L = 16
DT_STRIDE = (BATCH // 128) * 1024
WBUF = BT * 1024
NGROUP = PER_W // L
UNROLL = 16


def _emb_body(idx_hbm, tab_hbm, out_hbm, tab_v, idx_v, buf_v,
              sem_i, sem_o, sem_t):
    wid = lax.axis_index("s") * 2 + lax.axis_index("c")
    dt_w = wid // NG
    bw = wid % NG

    pltpu.async_copy(tab_hbm.at[pl.ds(dt_w * 8 * MAX_LEN, 8 * MAX_LEN)],
                     tab_v, sem_t).wait()

    def stage_idx(s, slot):
        pltpu.async_copy(
            idx_hbm.at[pl.ds(s // 8, 1), pl.ds(bw * BT, BT),
                       pl.ds(s % 8, 1), :],
            idx_v.at[slot], sem_i[slot])

    for slot in range(NBUF):
        stage_idx(slot, slot)

    def out_dst(s):
        return out_hbm.at[pl.ds(s, 1),
                          pl.ds(dt_w * DT_STRIDE + bw * WBUF, WBUF)]

    def loop_body(s, carry):
        for slot in range(NBUF):
            pltpu.make_async_copy(
                idx_hbm.at[pl.ds(0, 1), pl.ds(0, BT), pl.ds(0, 1), :],
                idx_v.at[slot], sem_i[slot]).wait()

            @pl.when(s >= NBUF)
            def _wait_prev():
                pltpu.make_async_copy(buf_v.at[slot], out_dst(0),
                                      sem_o[slot]).wait()

            def gather_group(i2, c):
                pend = []
                ids_list = [
                    idx_v[slot, 0, (i2 * UNROLL + u) // 8, 0,
                          pl.ds(((i2 * UNROLL + u) % 8) * L, L)]
                    for u in range(UNROLL)]
                for u in range(UNROLL):
                    i = i2 * UNROLL + u
                    ids = ids_list[u]
                    base = (i % 8) * L + (i // 8) * 1024
                    for dl in range(8):
                        v = plsc.load_gather(tab_v, [ids + dl * MAX_LEN])
                        if len(pend) >= 8:
                            pv, poff = pend.pop(0)
                            buf_v[slot, 0, pl.ds(poff, L)] = pv
                        pend.append((v, base + dl * 128))
                for pv, poff in pend:
                    buf_v[slot, 0, pl.ds(poff, L)] = pv
                return c

            lax.fori_loop(0, NGROUP // UNROLL, gather_group, 0)

            @pl.when(s + slot + NBUF < SEQ)
            def _stage_next():
                stage_idx(s + slot + NBUF, slot)

            pltpu.async_copy(buf_v.at[slot], out_dst(s + slot), sem_o[slot])
        return carry

    lax.fori_loop(0, SEQ // NBUF, lambda i, c: loop_body(i * NBUF, c), 0)

    for slot in range(NBUF):
        pltpu.make_async_copy(buf_v.at[slot], out_dst(0), sem_o[slot]).wait()


@jax.jit
def _emb(idx_t, tab_t):
    mesh = plsc.VectorSubcoreMesh(core_axis_name="c", subcore_axis_name="s")
    f = functools.partial(
        pl.kernel,
        mesh=mesh,
        out_type=jax.ShapeDtypeStruct((SEQ, DT * DT_STRIDE), jnp.float32),
        scratch_types=[
            pltpu.VMEM((8 * MAX_LEN,), jnp.float32),
            pltpu.VMEM((NBUF, 1, BT, 1, 128), jnp.int32),
            pltpu.VMEM((NBUF, 1, WBUF), jnp.float32),
            [pltpu.SemaphoreType.DMA] * NBUF,
            [pltpu.SemaphoreType.DMA] * NBUF,
            pltpu.SemaphoreType.DMA,
        ],
        compiler_params=pltpu.CompilerParams(use_tc_tiling_on_sc=False,
                                             needs_layout_passes=False),
    )(_emb_body)
    return f(idx_t, tab_t)


def kernel(residue_idx, embed_weight):
    idx_p = (residue_idx.astype(jnp.int32)
             .reshape(128, 128, 25, 8).transpose(2, 0, 3, 1))
    tab_t = embed_weight.T.reshape(-1)
    out2 = _emb(idx_p, tab_t)
    out5 = out2.reshape(SEQ, DT, BATCH // 128, 8, 128)
    return out5.transpose(2, 4, 0, 1, 3).reshape(BATCH, SEQ, EMBED_DIM)
